# Initial kernel scaffold; baseline (speedup 1.0000x reference)
import jax
import jax.numpy as jnp
from jax.experimental import pallas as pl


def kernel(emb_indices, table):
    raise NotImplementedError("write your pallas kernel here")


# speedup vs baseline: 3.5438x; 3.5438x over previous
import functools

import jax
import jax.numpy as jnp
from jax import lax
from jax.experimental import pallas as pl
from jax.experimental.pallas import tpu as pltpu
from jax.experimental.pallas import tpu_sc as plsc

_DIM = 64
_NW = 32
_CHUNK = 128


@functools.lru_cache(maxsize=None)
def _make_gather(n_chunk: int):
    mesh = plsc.VectorSubcoreMesh(core_axis_name="c", subcore_axis_name="s")
    rows_per_w = n_chunk * _CHUNK

    @functools.partial(
        pl.kernel,
        out_type=jax.ShapeDtypeStruct((_NW * rows_per_w, _DIM), jnp.float32),
        mesh=mesh,
        compiler_params=pltpu.CompilerParams(use_tc_tiling_on_sc=False),
        scratch_types=[
            pltpu.VMEM((n_chunk, _CHUNK), jnp.int32),
            pltpu.VMEM((_CHUNK, _DIM), jnp.float32),
            pltpu.SemaphoreType.DMA,
        ],
    )
    def k(idx_hbm, table_hbm, out_hbm, idx_v, rows_v, sem):
        wid = lax.axis_index("s") * 2 + lax.axis_index("c")
        base = wid * rows_per_w
        pltpu.sync_copy(idx_hbm.at[wid], idx_v)

        @pl.loop(0, n_chunk)
        def _(j):
            pltpu.async_copy(table_hbm.at[idx_v.at[j]], rows_v, sem).wait()
            pltpu.sync_copy(rows_v, out_hbm.at[pl.ds(base + j * _CHUNK, _CHUNK)])

    return k


def kernel(emb_indices, table):
    shape = emb_indices.shape
    idx = emb_indices.reshape(_NW, -1, _CHUNK)
    out = _make_gather(idx.shape[1])(idx, table)
    return out.reshape(*shape, _DIM)


# speedup vs baseline: 4.2647x; 1.2034x over previous
import functools

import jax
import jax.numpy as jnp
from jax import lax
from jax.experimental import pallas as pl
from jax.experimental.pallas import tpu as pltpu
from jax.experimental.pallas import tpu_sc as plsc

_DIM = 64
_NW = 32
_CHUNK = 128
_NBUF = 8


@functools.lru_cache(maxsize=None)
def _make_gather(n_chunk: int):
    mesh = plsc.VectorSubcoreMesh(core_axis_name="c", subcore_axis_name="s")
    rows_per_w = n_chunk * _CHUNK
    n_grp = n_chunk // _NBUF

    @functools.partial(
        pl.kernel,
        out_type=jax.ShapeDtypeStruct((_NW * rows_per_w, _DIM), jnp.float32),
        mesh=mesh,
        compiler_params=pltpu.CompilerParams(use_tc_tiling_on_sc=False),
        scratch_types=[pltpu.VMEM((n_chunk, _CHUNK), jnp.int32)]
        + [pltpu.VMEM((_CHUNK, _DIM), jnp.float32)] * _NBUF
        + [pltpu.SemaphoreType.DMA] * (2 * _NBUF),
    )
    def k(idx_hbm, table_hbm, out_hbm, idx_v, *rest):
        rows = rest[:_NBUF]
        gsem = rest[_NBUF:2 * _NBUF]
        osem = rest[2 * _NBUF:]
        wid = lax.axis_index("s") * 2 + lax.axis_index("c")
        base = wid * rows_per_w
        pltpu.sync_copy(idx_hbm.at[wid], idx_v)

        def fire_gather(b, j):
            pltpu.async_copy(table_hbm.at[idx_v.at[j]], rows[b], gsem[b])

        def wait_gather(b, j):
            pltpu.make_async_copy(table_hbm.at[idx_v.at[j]], rows[b], gsem[b]).wait()

        def fire_out(b, j):
            pltpu.async_copy(rows[b], out_hbm.at[pl.ds(base + j * _CHUNK, _CHUNK)], osem[b])

        def wait_out(b, j):
            pltpu.make_async_copy(rows[b], out_hbm.at[pl.ds(base + j * _CHUNK, _CHUNK)], osem[b]).wait()

        for b in range(_NBUF):
            fire_gather(b, b)

        @pl.loop(0, n_grp - 1)
        def _(g):
            j0 = g * _NBUF
            for b in range(_NBUF):
                wait_gather(b, j0 + b)
                fire_out(b, j0 + b)
            for b in range(_NBUF):
                wait_out(b, j0 + b)
                fire_gather(b, j0 + _NBUF + b)

        j0 = (n_grp - 1) * _NBUF
        for b in range(_NBUF):
            wait_gather(b, j0 + b)
            fire_out(b, j0 + b)
        for b in range(_NBUF):
            wait_out(b, j0 + b)

    return k


def kernel(emb_indices, table):
    shape = emb_indices.shape
    idx = emb_indices.reshape(_NW, -1, _CHUNK)
    out = _make_gather(idx.shape[1])(idx, table)
    return out.reshape(*shape, _DIM)
